# baseline (speedup 1.0000x reference)
import functools

import jax
import jax.numpy as jnp
import numpy as np
from jax import lax
from jax.experimental import pallas as pl
from jax.experimental.pallas import tpu as pltpu
import jax.experimental.pallas.tpu_sc as plsc

CUTOFF = 5.0
WIDTH = 0.5
N_RADIAL = 9
N_PSEUDO = 4
N_NODES = 10000
N_EDGES = 160000
N_SPECIES = 8
MAX_L = 3
NK = 144
NGROUP = 9
GW = NK // NGROUP
NSEG = N_NODES * N_SPECIES


def _jl_np(l, x):
    x = np.asarray(x, dtype=np.float64)
    s = np.sin(x); c = np.cos(x)
    if l == 0:
        return s / x
    if l == 1:
        return s / x**2 - c / x
    if l == 2:
        return (3.0 / x**3 - 1.0 / x) * s - 3.0 * c / x**2
    return (15.0 / x**4 - 6.0 / x**2) * s - (15.0 / x**3 - 1.0 / x) * c


def _bessel_zeros(l, n):
    xs = np.linspace(0.1, 60.0, 120001)
    v = _jl_np(l, xs)
    idx = np.nonzero(np.sign(v[:-1]) * np.sign(v[1:]) < 0)[0][:n]
    roots = []
    for k in idx:
        a, b = float(xs[k]), float(xs[k + 1])
        fa = float(_jl_np(l, a))
        for _ in range(60):
            m = 0.5 * (a + b)
            fm = float(_jl_np(l, m))
            if fa * fm <= 0.0:
                b = m
            else:
                a, fa = m, fm
        roots.append(0.5 * (a + b))
    return np.asarray(roots)


_ZEROS = np.stack([_bessel_zeros(l, N_RADIAL) for l in range(MAX_L + 1)])
_ZFLAT = _ZEROS.reshape(1, 4 * N_RADIAL).astype(np.float32)

_LOFF = [0, 9, 36, 81]
_NRAD = 4 * N_RADIAL

_SELY = np.zeros((16, NK), np.float32)
_SELR = np.zeros((_NRAD, NK), np.float32)
_lmoff = [0, 1, 4, 9]
for _l in range(MAX_L + 1):
    for _m in range(2 * _l + 1):
        for _n in range(N_RADIAL):
            _k = _LOFF[_l] + _m * N_RADIAL + _n
            _SELY[_lmoff[_l] + _m, _k] = 1.0
            _SELR[_l * N_RADIAL + _n, _k] = 1.0


_AROWS = N_EDGES // 128
_ABR = 10
_AEB = _ABR * 128


def _edge_feat_body(x_ref, y_ref, z_ref, zcol_ref, sely_ref, selr_ref,
                    out_ref):
    xs = x_ref[0]
    ys = y_ref[0]
    zs = z_ref[0]
    r2 = xs * xs + ys * ys + zs * zs + 1e-20
    r = jnp.sqrt(r2)
    ux = xs / r
    uy = ys / r
    uz = zs / r
    t = jnp.clip((r - (CUTOFF - WIDTH)) / WIDTH, 0.0, 1.0)
    fc = 0.5 * (1.0 + jnp.cos(np.pi * t))
    xx = r / CUTOFF
    zc = zcol_ref[...]

    for rb in range(_ABR):
        x1 = ux[rb:rb + 1, :]
        y1 = uy[rb:rb + 1, :]
        z1 = uz[rb:rb + 1, :]
        one = jnp.ones_like(x1)
        yrows = [
            0.28209479177387814 * one,
            0.4886025119029199 * y1,
            0.4886025119029199 * z1,
            0.4886025119029199 * x1,
            1.0925484305920792 * x1 * y1,
            1.0925484305920792 * y1 * z1,
            0.31539156525252005 * (3.0 * z1 * z1 - 1.0),
            1.0925484305920792 * x1 * z1,
            0.5462742152960396 * (x1 * x1 - y1 * y1),
            0.5900435899266435 * y1 * (3.0 * x1 * x1 - y1 * y1),
            2.890611442640554 * x1 * y1 * z1,
            0.4570457994644658 * y1 * (5.0 * z1 * z1 - 1.0),
            0.3731763325901154 * z1 * (5.0 * z1 * z1 - 3.0),
            0.4570457994644658 * x1 * (5.0 * z1 * z1 - 1.0),
            1.445305721320277 * z1 * (x1 * x1 - y1 * y1),
            0.5900435899266435 * x1 * (x1 * x1 - 3.0 * y1 * y1),
        ]
        Yt = jnp.concatenate(yrows, axis=0)

        arg = jnp.maximum(zc * xx[rb:rb + 1, :], 1e-6)
        s = jnp.sin(arg)
        c = jnp.cos(arg)
        a0 = arg[0:9, :]
        s0 = s[0:9, :]
        a1 = arg[9:18, :]
        s1 = s[9:18, :]
        c1 = c[9:18, :]
        a2 = arg[18:27, :]
        s2 = s[18:27, :]
        c2 = c[18:27, :]
        a3 = arg[27:36, :]
        s3 = s[27:36, :]
        c3 = c[27:36, :]
        j0 = s0 / a0
        j1 = s1 / a1**2 - c1 / a1
        j2 = (3.0 / a2**3 - 1.0 / a2) * s2 - 3.0 * c2 / a2**2
        j3 = ((15.0 / a3**4 - 6.0 / a3**2) * s3
              - (15.0 / a3**3 - 1.0 / a3) * c3)
        Rt = jnp.concatenate([j0, j1, j2, j3], axis=0) * fc[rb:rb + 1, :]

        Ye = Yt.T
        Re = Rt.T
        G = (jnp.dot(Ye, sely_ref[...], preferred_element_type=jnp.float32)
             * jnp.dot(Re, selr_ref[...],
                       preferred_element_type=jnp.float32))
        for g in range(NGROUP):
            out_ref[g, rb * 128:(rb + 1) * 128, :] = \
                G[:, g * GW:(g + 1) * GW]


def _edge_features(R_ij):
    grid = _AROWS // _ABR
    planes = R_ij.T.reshape(3, grid, _ABR, 128)
    out = pl.pallas_call(
        _edge_feat_body,
        grid=(grid,),
        in_specs=[
            pl.BlockSpec((1, _ABR, 128), lambda ib: (ib, 0, 0)),
            pl.BlockSpec((1, _ABR, 128), lambda ib: (ib, 0, 0)),
            pl.BlockSpec((1, _ABR, 128), lambda ib: (ib, 0, 0)),
            pl.BlockSpec((_NRAD, 1), lambda ib: (0, 0)),
            pl.BlockSpec((16, NK), lambda ib: (0, 0)),
            pl.BlockSpec((_NRAD, NK), lambda ib: (0, 0)),
        ],
        out_specs=pl.BlockSpec((NGROUP, _AEB, GW), lambda ib: (0, ib, 0)),
        out_shape=jax.ShapeDtypeStruct((NGROUP, N_EDGES, GW), jnp.float32),
    )(planes[0], planes[1], planes[2],
      jnp.asarray(_ZFLAT.reshape(_NRAD, 1)), jnp.asarray(_SELY),
      jnp.asarray(_SELR))
    return out.reshape(NGROUP * N_EDGES, GW)


_SC_NC = 2
_SC_NT = 16
_PASSES = -(-NGROUP // _SC_NC)
_CHUNK = 128
_NCHUNKS = N_EDGES // _CHUNK
_ROWS_PT = NSEG // _SC_NT
_DCH = 1000


def _sc_scatter(g2, i, j, species, zrows):
    mesh = plsc.VectorSubcoreMesh(core_axis_name="c", subcore_axis_name="s")

    @functools.partial(
        pl.kernel,
        out_type=jax.ShapeDtypeStruct((NGROUP * NSEG, GW), jnp.float32),
        mesh=mesh,
        compiler_params=pltpu.CompilerParams(
            needs_layout_passes=False, use_tc_tiling_on_sc=False),
        scratch_types=[
            pltpu.VMEM_SHARED((NSEG, GW), jnp.float32),
            pltpu.VMEM((N_NODES,), jnp.int32),
            pltpu.VMEM((_DCH, GW), jnp.float32),
            pltpu.VMEM((_CHUNK, GW), jnp.float32),
            pltpu.VMEM((_CHUNK, GW), jnp.float32),
            pltpu.VMEM((_CHUNK,), jnp.int32),
            pltpu.VMEM((_CHUNK,), jnp.int32),
            pltpu.VMEM((_CHUNK,), jnp.int32),
            pltpu.VMEM((_CHUNK,), jnp.int32),
            pltpu.VMEM((_CHUNK,), jnp.int32),
            pltpu.VMEM((_CHUNK,), jnp.int32),
            pltpu.SemaphoreType.DMA,
            pltpu.SemaphoreType.DMA,
        ],
    )
    def scatter_kernel(g_hbm, i_hbm, j_hbm, sp_hbm, z_hbm, h_hbm,
                       slab, spec_v, zbuf, gb0, gb1, ib0, ib1, jb0, jb1,
                       kb0, kb1, sem0, sem1):
        c = lax.axis_index("c")
        t = lax.axis_index("s")
        gbuf = (gb0, gb1)
        ibuf = (ib0, ib1)
        jbuf = (jb0, jb1)
        kbuf = (kb0, kb1)
        sems = (sem0, sem1)

        pltpu.sync_copy(sp_hbm, spec_v)
        pltpu.sync_copy(z_hbm, zbuf)

        q0 = (t * _NCHUNKS) // _SC_NT
        q1 = ((t + 1) * _NCHUNKS) // _SC_NT
        nq = q1 - q0
        r0 = t * _ROWS_PT

        def fire(grow_base, q, slot):
            e0 = pl.multiple_of(q * _CHUNK, _CHUNK)
            pltpu.async_copy(
                g_hbm.at[pl.ds(grow_base + e0, _CHUNK), :], gbuf[slot],
                sems[slot])
            pltpu.async_copy(i_hbm.at[pl.ds(e0, _CHUNK)], ibuf[slot],
                             sems[slot])
            pltpu.async_copy(j_hbm.at[pl.ds(e0, _CHUNK)], jbuf[slot],
                             sems[slot])

        def drain(slot):
            pltpu.make_async_copy(
                g_hbm.at[pl.ds(0, _CHUNK), :], gbuf[slot], sems[slot]).wait()
            pltpu.make_async_copy(
                i_hbm.at[pl.ds(0, _CHUNK)], ibuf[slot], sems[slot]).wait()
            pltpu.make_async_copy(
                j_hbm.at[pl.ds(0, _CHUNK)], jbuf[slot], sems[slot]).wait()

        for p in range(_PASSES):
            g = p * _SC_NC + c

            @pl.when(g < NGROUP)
            def _pass_body(g=g):
                grow_base = g * N_EDGES
                hrow_base = g * NSEG

                for dch in range(_ROWS_PT // _DCH):
                    zoff = pl.multiple_of(r0 + dch * _DCH, 8)
                    pltpu.sync_copy(zbuf, slab.at[pl.ds(zoff, _DCH), :])
                plsc.subcore_barrier()

                def chunk_iter(s, slot):
                    drain(slot)

                    @pl.when(s + 1 < nq)
                    def _():
                        fire(grow_base, q0 + s + 1, 1 - slot)

                    for u in range(_CHUNK // 16):
                        iv = ibuf[slot][pl.ds(u * 16, 16)]
                        jv = jbuf[slot][pl.ds(u * 16, 16)]
                        sv = plsc.load_gather(spec_v, [jv])
                        kbuf[slot][pl.ds(u * 16, 16)] = iv * N_SPECIES + sv
                    pltpu.sync_copy(gbuf[slot], slab.at[kbuf[slot]],
                                    add=True)

                fire(grow_base, q0, 0)

                def loop_body(s, carry):
                    even = lax.rem(s, 2) == 0

                    @pl.when(even)
                    def _():
                        chunk_iter(s, 0)

                    @pl.when(jnp.logical_not(even))
                    def _():
                        chunk_iter(s, 1)

                    return carry

                lax.fori_loop(0, nq, loop_body, 0)
                plsc.subcore_barrier()

                for dch in range(_ROWS_PT // _DCH):
                    rr = pl.multiple_of(r0 + dch * _DCH, 8)
                    ro = pl.multiple_of(hrow_base + r0 + dch * _DCH, 8)
                    pltpu.sync_copy(slab.at[pl.ds(rr, _DCH), :],
                                    h_hbm.at[pl.ds(ro, _DCH), :])

    return scatter_kernel(g2, i, j, species, zrows)


def _segment_accumulate(g2, i, j, species):
    zrows = jnp.zeros((_DCH, GW), jnp.float32)
    return _sc_scatter(g2, i.astype(jnp.int32), j.astype(jnp.int32),
                       species.astype(jnp.int32), zrows)


_NB = 128

_TREP = np.zeros((36, 1296), np.float32)
_TTILE = np.zeros((36, 1296), np.float32)
for _a in range(36):
    for _b in range(36):
        _TREP[_a, _a * 36 + _b] = 1.0
        _TTILE[_b, _a * 36 + _b] = 1.0

_lm_of = []
for _l in range(MAX_L + 1):
    for _m in range(2 * _l + 1):
        _lm_of.append((_l, _m))

_STRUCT = np.zeros((NGROUP * N_SPECIES * GW, 16 * 36), np.float32)
_SROW = np.zeros((NGROUP * N_SPECIES * GW,), np.int64)
_PCOL = np.zeros((16 * 36,), np.int64)
for _g in range(NGROUP):
    for _s in range(N_SPECIES):
        for _k in range(GW):
            _row = _g * 128 + _s * GW + _k
            _SROW[_row] = _s
            _kg = _g * GW + _k
            for _lm in range(16):
                _l, _m = _lm_of[_lm]
                _n = _kg - (_LOFF[_l] + _m * N_RADIAL)
                if 0 <= _n < N_RADIAL:
                    for _p in range(N_PSEUDO):
                        _col = _lm * 36 + _p * N_RADIAL + _n
                        _STRUCT[_row, _col] = 1.0
                        _PCOL[_col] = _p


def _species_matrix(W_species):
    wexp = W_species[_SROW][:, _PCOL]
    return wexp * jnp.asarray(_STRUCT)


def _power_body(w_ref, m2_ref, trep_ref, ttile_ref, *refs):
    h_refs = refs[:NGROUP]
    out_ref = refs[NGROUP]
    del w_ref
    Hcat = jnp.concatenate([h[0] for h in h_refs], axis=1)
    V_all = jnp.dot(Hcat, m2_ref[...],
                    precision=lax.Precision.HIGHEST,
                    preferred_element_type=jnp.float32)

    Tr = trep_ref[...]
    Tt = ttile_ref[...]
    vst = []
    for l in range(MAX_L + 1):
        lm0 = _lmoff[l]
        vst.append(jnp.concatenate(
            [V_all[:, (lm0 + m) * 36:(lm0 + m + 1) * 36]
             for m in range(2 * l + 1)], axis=0))
    vrs = [jnp.dot(v, Tr, preferred_element_type=jnp.float32) for v in vst]
    vts = [jnp.dot(v, Tt, preferred_element_type=jnp.float32) for v in vst]
    outcols = []
    for l in range(MAX_L + 1):
        prod = vrs[l] * vts[l]
        acc = prod.reshape(2 * l + 1, _NB, 1296).sum(axis=0)
        outcols.append(acc)
    out_ref[...] = jnp.concatenate(outcols, axis=1)


def _power_spectrum_tc(h2, W_species):
    h3 = h2.reshape(NGROUP, N_NODES, N_SPECIES * GW)
    grid = pl.cdiv(N_NODES, _NB)

    def _hmap(g):
        return lambda ib: (g, ib, 0)

    return pl.pallas_call(
        _power_body,
        grid=(grid,),
        in_specs=[pl.BlockSpec(memory_space=pltpu.SMEM),
                  pl.BlockSpec((NGROUP * 128, 16 * 36), lambda ib: (0, 0)),
                  pl.BlockSpec((36, 1296), lambda ib: (0, 0)),
                  pl.BlockSpec((36, 1296), lambda ib: (0, 0))]
        + [pl.BlockSpec((1, _NB, N_SPECIES * GW), _hmap(g))
           for g in range(NGROUP)],
        out_specs=pl.BlockSpec((_NB, 36 * 36 * 4), lambda ib: (ib, 0)),
        out_shape=jax.ShapeDtypeStruct((N_NODES, 36 * 36 * 4), jnp.float32),
    )(W_species, _species_matrix(W_species), jnp.asarray(_TREP),
      jnp.asarray(_TTILE), *([h3] * NGROUP))


def kernel(R_ij, i, j, species, structures, centers, W_species):
    g2 = _edge_features(R_ij)
    h2 = _segment_accumulate(g2, i, j, species)
    return _power_spectrum_tc(h2, W_species)


# speedup vs baseline: 1.0250x; 1.0250x over previous
import functools

import jax
import jax.numpy as jnp
import numpy as np
from jax import lax
from jax.experimental import pallas as pl
from jax.experimental.pallas import tpu as pltpu
import jax.experimental.pallas.tpu_sc as plsc

CUTOFF = 5.0
WIDTH = 0.5
N_RADIAL = 9
N_PSEUDO = 4
N_NODES = 10000
N_EDGES = 160000
N_SPECIES = 8
MAX_L = 3
NK = 144
NGROUP = 9
GW = NK // NGROUP
NSEG = N_NODES * N_SPECIES


def _jl_np(l, x):
    x = np.asarray(x, dtype=np.float64)
    s = np.sin(x); c = np.cos(x)
    if l == 0:
        return s / x
    if l == 1:
        return s / x**2 - c / x
    if l == 2:
        return (3.0 / x**3 - 1.0 / x) * s - 3.0 * c / x**2
    return (15.0 / x**4 - 6.0 / x**2) * s - (15.0 / x**3 - 1.0 / x) * c


def _bessel_zeros(l, n):
    xs = np.linspace(0.1, 60.0, 120001)
    v = _jl_np(l, xs)
    idx = np.nonzero(np.sign(v[:-1]) * np.sign(v[1:]) < 0)[0][:n]
    roots = []
    for k in idx:
        a, b = float(xs[k]), float(xs[k + 1])
        fa = float(_jl_np(l, a))
        for _ in range(60):
            m = 0.5 * (a + b)
            fm = float(_jl_np(l, m))
            if fa * fm <= 0.0:
                b = m
            else:
                a, fa = m, fm
        roots.append(0.5 * (a + b))
    return np.asarray(roots)


_ZEROS = np.stack([_bessel_zeros(l, N_RADIAL) for l in range(MAX_L + 1)])
_ZFLAT = _ZEROS.reshape(1, 4 * N_RADIAL).astype(np.float32)

_LOFF = [0, 9, 36, 81]
_NRAD = 4 * N_RADIAL

_SELY = np.zeros((16, NK), np.float32)
_SELR = np.zeros((_NRAD, NK), np.float32)
_lmoff = [0, 1, 4, 9]
for _l in range(MAX_L + 1):
    for _m in range(2 * _l + 1):
        for _n in range(N_RADIAL):
            _k = _LOFF[_l] + _m * N_RADIAL + _n
            _SELY[_lmoff[_l] + _m, _k] = 1.0
            _SELR[_l * N_RADIAL + _n, _k] = 1.0


_AROWS = N_EDGES // 128
_ABR = 10
_AEB = _ABR * 128


def _edge_feat_body(x_ref, y_ref, z_ref, zcol_ref, sely_ref, selr_ref,
                    out_ref):
    xs = x_ref[0]
    ys = y_ref[0]
    zs = z_ref[0]
    r2 = xs * xs + ys * ys + zs * zs + 1e-20
    r = jnp.sqrt(r2)
    ux = xs / r
    uy = ys / r
    uz = zs / r
    t = jnp.clip((r - (CUTOFF - WIDTH)) / WIDTH, 0.0, 1.0)
    fc = 0.5 * (1.0 + jnp.cos(np.pi * t))
    xx = r / CUTOFF
    zc = zcol_ref[...]

    for rb in range(_ABR):
        x1 = ux[rb:rb + 1, :]
        y1 = uy[rb:rb + 1, :]
        z1 = uz[rb:rb + 1, :]
        one = jnp.ones_like(x1)
        yrows = [
            0.28209479177387814 * one,
            0.4886025119029199 * y1,
            0.4886025119029199 * z1,
            0.4886025119029199 * x1,
            1.0925484305920792 * x1 * y1,
            1.0925484305920792 * y1 * z1,
            0.31539156525252005 * (3.0 * z1 * z1 - 1.0),
            1.0925484305920792 * x1 * z1,
            0.5462742152960396 * (x1 * x1 - y1 * y1),
            0.5900435899266435 * y1 * (3.0 * x1 * x1 - y1 * y1),
            2.890611442640554 * x1 * y1 * z1,
            0.4570457994644658 * y1 * (5.0 * z1 * z1 - 1.0),
            0.3731763325901154 * z1 * (5.0 * z1 * z1 - 3.0),
            0.4570457994644658 * x1 * (5.0 * z1 * z1 - 1.0),
            1.445305721320277 * z1 * (x1 * x1 - y1 * y1),
            0.5900435899266435 * x1 * (x1 * x1 - 3.0 * y1 * y1),
        ]
        Yt = jnp.concatenate(yrows, axis=0)

        arg = jnp.maximum(zc * xx[rb:rb + 1, :], 1e-6)
        s = jnp.sin(arg)
        c = jnp.cos(arg)
        a0 = arg[0:9, :]
        s0 = s[0:9, :]
        a1 = arg[9:18, :]
        s1 = s[9:18, :]
        c1 = c[9:18, :]
        a2 = arg[18:27, :]
        s2 = s[18:27, :]
        c2 = c[18:27, :]
        a3 = arg[27:36, :]
        s3 = s[27:36, :]
        c3 = c[27:36, :]
        j0 = s0 / a0
        j1 = s1 / a1**2 - c1 / a1
        j2 = (3.0 / a2**3 - 1.0 / a2) * s2 - 3.0 * c2 / a2**2
        j3 = ((15.0 / a3**4 - 6.0 / a3**2) * s3
              - (15.0 / a3**3 - 1.0 / a3) * c3)
        Rt = jnp.concatenate([j0, j1, j2, j3], axis=0) * fc[rb:rb + 1, :]

        Ye = Yt.T
        Re = Rt.T
        G = (jnp.dot(Ye, sely_ref[...], preferred_element_type=jnp.float32)
             * jnp.dot(Re, selr_ref[...],
                       preferred_element_type=jnp.float32))
        for g in range(NGROUP):
            out_ref[g, rb * 128:(rb + 1) * 128, :] = \
                G[:, g * GW:(g + 1) * GW]


def _edge_features(R_ij):
    grid = _AROWS // _ABR
    planes = R_ij.T.reshape(3, grid, _ABR, 128)
    out = pl.pallas_call(
        _edge_feat_body,
        grid=(grid,),
        in_specs=[
            pl.BlockSpec((1, _ABR, 128), lambda ib: (ib, 0, 0)),
            pl.BlockSpec((1, _ABR, 128), lambda ib: (ib, 0, 0)),
            pl.BlockSpec((1, _ABR, 128), lambda ib: (ib, 0, 0)),
            pl.BlockSpec((_NRAD, 1), lambda ib: (0, 0)),
            pl.BlockSpec((16, NK), lambda ib: (0, 0)),
            pl.BlockSpec((_NRAD, NK), lambda ib: (0, 0)),
        ],
        out_specs=pl.BlockSpec((NGROUP, _AEB, GW), lambda ib: (0, ib, 0)),
        out_shape=jax.ShapeDtypeStruct((NGROUP, N_EDGES, GW), jnp.float32),
    )(planes[0], planes[1], planes[2],
      jnp.asarray(_ZFLAT.reshape(_NRAD, 1)), jnp.asarray(_SELY),
      jnp.asarray(_SELR))
    return out.reshape(NGROUP * N_EDGES, GW)


_SC_NC = 2
_SC_NT = 16
_PASSES = -(-NGROUP // _SC_NC)
_CHUNK = 128
_NCHUNKS = N_EDGES // _CHUNK
_ROWS_PT = NSEG // _SC_NT
_DCH = 1000


def _sc_scatter(g2, i, j, species, zrows):
    mesh = plsc.VectorSubcoreMesh(core_axis_name="c", subcore_axis_name="s")

    @functools.partial(
        pl.kernel,
        out_type=jax.ShapeDtypeStruct((NGROUP * NSEG, GW), jnp.float32),
        mesh=mesh,
        compiler_params=pltpu.CompilerParams(
            needs_layout_passes=False, use_tc_tiling_on_sc=False),
        scratch_types=[
            pltpu.VMEM_SHARED((NSEG, GW), jnp.float32),
            pltpu.VMEM((N_NODES,), jnp.int32),
            pltpu.VMEM((_DCH, GW), jnp.float32),
            pltpu.VMEM((_CHUNK, GW), jnp.float32),
            pltpu.VMEM((_CHUNK, GW), jnp.float32),
            pltpu.VMEM((_CHUNK,), jnp.int32),
            pltpu.VMEM((_CHUNK,), jnp.int32),
            pltpu.VMEM((_CHUNK,), jnp.int32),
            pltpu.VMEM((_CHUNK,), jnp.int32),
            pltpu.VMEM((_CHUNK,), jnp.int32),
            pltpu.VMEM((_CHUNK,), jnp.int32),
            pltpu.SemaphoreType.DMA,
            pltpu.SemaphoreType.DMA,
        ],
    )
    def scatter_kernel(g_hbm, i_hbm, j_hbm, sp_hbm, z_hbm, h_hbm,
                       slab, spec_v, zbuf, gb0, gb1, ib0, ib1, jb0, jb1,
                       kb0, kb1, sem0, sem1):
        c = lax.axis_index("c")
        t = lax.axis_index("s")
        gbuf = (gb0, gb1)
        ibuf = (ib0, ib1)
        jbuf = (jb0, jb1)
        kbuf = (kb0, kb1)
        sems = (sem0, sem1)

        pltpu.sync_copy(sp_hbm, spec_v)
        pltpu.sync_copy(z_hbm, zbuf)

        q0 = (t * _NCHUNKS) // _SC_NT
        q1 = ((t + 1) * _NCHUNKS) // _SC_NT
        nq = q1 - q0
        r0 = t * _ROWS_PT

        def fire(grow_base, q, slot):
            e0 = pl.multiple_of(q * _CHUNK, _CHUNK)
            pltpu.async_copy(
                g_hbm.at[pl.ds(grow_base + e0, _CHUNK), :], gbuf[slot],
                sems[slot])
            pltpu.async_copy(i_hbm.at[pl.ds(e0, _CHUNK)], ibuf[slot],
                             sems[slot])
            pltpu.async_copy(j_hbm.at[pl.ds(e0, _CHUNK)], jbuf[slot],
                             sems[slot])

        def drain(slot):
            pltpu.make_async_copy(
                g_hbm.at[pl.ds(0, _CHUNK), :], gbuf[slot], sems[slot]).wait()
            pltpu.make_async_copy(
                i_hbm.at[pl.ds(0, _CHUNK)], ibuf[slot], sems[slot]).wait()
            pltpu.make_async_copy(
                j_hbm.at[pl.ds(0, _CHUNK)], jbuf[slot], sems[slot]).wait()

        for p in range(_PASSES):
            g = p * _SC_NC + c

            @pl.when(g < NGROUP)
            def _pass_body(g=g):
                grow_base = g * N_EDGES
                hrow_base = g * NSEG

                for dch in range(_ROWS_PT // _DCH):
                    zoff = pl.multiple_of(r0 + dch * _DCH, 8)
                    pltpu.sync_copy(zbuf, slab.at[pl.ds(zoff, _DCH), :])
                plsc.subcore_barrier()

                def chunk_iter(s, slot):
                    drain(slot)

                    @pl.when(s + 1 < nq)
                    def _():
                        fire(grow_base, q0 + s + 1, 1 - slot)

                    for u in range(_CHUNK // 16):
                        iv = ibuf[slot][pl.ds(u * 16, 16)]
                        jv = jbuf[slot][pl.ds(u * 16, 16)]
                        sv = plsc.load_gather(spec_v, [jv])
                        kbuf[slot][pl.ds(u * 16, 16)] = iv * N_SPECIES + sv
                    pltpu.sync_copy(gbuf[slot], slab.at[kbuf[slot]],
                                    add=True)

                fire(grow_base, q0, 0)

                def loop_body(s, carry):
                    even = lax.rem(s, 2) == 0

                    @pl.when(even)
                    def _():
                        chunk_iter(s, 0)

                    @pl.when(jnp.logical_not(even))
                    def _():
                        chunk_iter(s, 1)

                    return carry

                lax.fori_loop(0, nq, loop_body, 0)
                plsc.subcore_barrier()

                for dch in range(_ROWS_PT // _DCH):
                    rr = pl.multiple_of(r0 + dch * _DCH, 8)
                    ro = pl.multiple_of(hrow_base + r0 + dch * _DCH, 8)
                    pltpu.sync_copy(slab.at[pl.ds(rr, _DCH), :],
                                    h_hbm.at[pl.ds(ro, _DCH), :])

    return scatter_kernel(g2, i, j, species, zrows)


def _segment_accumulate(g2, i, j, species):
    zrows = jnp.zeros((_DCH, GW), jnp.float32)
    return _sc_scatter(g2, i.astype(jnp.int32), j.astype(jnp.int32),
                       species.astype(jnp.int32), zrows)


_NB = 128

_TREP = np.zeros((36, 1296), np.float32)
_TTILE = np.zeros((36, 1296), np.float32)
for _a in range(36):
    for _b in range(36):
        _TREP[_a, _a * 36 + _b] = 1.0
        _TTILE[_b, _a * 36 + _b] = 1.0

_lm_of = []
for _l in range(MAX_L + 1):
    for _m in range(2 * _l + 1):
        _lm_of.append((_l, _m))

_TSEL = np.zeros((N_PSEUDO, NGROUP * N_SPECIES * GW, 16 * 36), np.float32)
for _g in range(NGROUP):
    for _s in range(N_SPECIES):
        for _k in range(GW):
            _row = _g * 128 + _s * GW + _k
            _kg = _g * GW + _k
            for _lm in range(16):
                _l, _m = _lm_of[_lm]
                _n = _kg - (_LOFF[_l] + _m * N_RADIAL)
                if 0 <= _n < N_RADIAL:
                    for _p in range(N_PSEUDO):
                        _TSEL[_p, _row, _lm * 36 + _p * N_RADIAL + _n] = 1.0


def _species_lane_weights(W_species):
    return jnp.stack([
        jnp.tile(jnp.repeat(W_species[:, p], GW), NGROUP)
        for p in range(N_PSEUDO)])


def _power_body(w_ref, wvec_ref, tsel_ref, trep_ref, ttile_ref, *refs):
    h_refs = refs[:NGROUP]
    out_ref = refs[NGROUP]
    del w_ref
    Hcat = jnp.concatenate([h[0] for h in h_refs], axis=1)
    V_all = None
    for p in range(N_PSEUDO):
        hw = Hcat * wvec_ref[p:p + 1, :]
        part = jnp.dot(hw, tsel_ref[p],
                       preferred_element_type=jnp.float32)
        V_all = part if V_all is None else V_all + part

    Tr = trep_ref[...]
    Tt = ttile_ref[...]
    vst = []
    for l in range(MAX_L + 1):
        lm0 = _lmoff[l]
        vst.append(jnp.concatenate(
            [V_all[:, (lm0 + m) * 36:(lm0 + m + 1) * 36]
             for m in range(2 * l + 1)], axis=0))
    vrs = [jnp.dot(v, Tr, preferred_element_type=jnp.float32) for v in vst]
    vts = [jnp.dot(v, Tt, preferred_element_type=jnp.float32) for v in vst]
    outcols = []
    for l in range(MAX_L + 1):
        prod = vrs[l] * vts[l]
        acc = prod.reshape(2 * l + 1, _NB, 1296).sum(axis=0)
        outcols.append(acc)
    out_ref[...] = jnp.concatenate(outcols, axis=1)


def _power_spectrum_tc(h2, W_species):
    h3 = h2.reshape(NGROUP, N_NODES, N_SPECIES * GW)
    grid = pl.cdiv(N_NODES, _NB)

    def _hmap(g):
        return lambda ib: (g, ib, 0)

    return pl.pallas_call(
        _power_body,
        grid=(grid,),
        in_specs=[pl.BlockSpec(memory_space=pltpu.SMEM),
                  pl.BlockSpec((N_PSEUDO, NGROUP * 128), lambda ib: (0, 0)),
                  pl.BlockSpec((N_PSEUDO, NGROUP * 128, 16 * 36),
                               lambda ib: (0, 0, 0)),
                  pl.BlockSpec((36, 1296), lambda ib: (0, 0)),
                  pl.BlockSpec((36, 1296), lambda ib: (0, 0))]
        + [pl.BlockSpec((1, _NB, N_SPECIES * GW), _hmap(g))
           for g in range(NGROUP)],
        out_specs=pl.BlockSpec((_NB, 36 * 36 * 4), lambda ib: (ib, 0)),
        out_shape=jax.ShapeDtypeStruct((N_NODES, 36 * 36 * 4), jnp.float32),
    )(W_species, _species_lane_weights(W_species), jnp.asarray(_TSEL),
      jnp.asarray(_TREP), jnp.asarray(_TTILE), *([h3] * NGROUP))


def kernel(R_ij, i, j, species, structures, centers, W_species):
    g2 = _edge_features(R_ij)
    h2 = _segment_accumulate(g2, i, j, species)
    return _power_spectrum_tc(h2, W_species)


# speedup vs baseline: 1.3471x; 1.3143x over previous
import functools

import jax
import jax.numpy as jnp
import numpy as np
from jax import lax
from jax.experimental import pallas as pl
from jax.experimental.pallas import tpu as pltpu
import jax.experimental.pallas.tpu_sc as plsc

CUTOFF = 5.0
WIDTH = 0.5
N_RADIAL = 9
N_PSEUDO = 4
N_NODES = 10000
N_EDGES = 160000
N_SPECIES = 8
MAX_L = 3
NK = 144
NGROUP = 9
GW = NK // NGROUP
NSEG = N_NODES * N_SPECIES


def _jl_np(l, x):
    x = np.asarray(x, dtype=np.float64)
    s = np.sin(x); c = np.cos(x)
    if l == 0:
        return s / x
    if l == 1:
        return s / x**2 - c / x
    if l == 2:
        return (3.0 / x**3 - 1.0 / x) * s - 3.0 * c / x**2
    return (15.0 / x**4 - 6.0 / x**2) * s - (15.0 / x**3 - 1.0 / x) * c


def _bessel_zeros(l, n):
    xs = np.linspace(0.1, 60.0, 120001)
    v = _jl_np(l, xs)
    idx = np.nonzero(np.sign(v[:-1]) * np.sign(v[1:]) < 0)[0][:n]
    roots = []
    for k in idx:
        a, b = float(xs[k]), float(xs[k + 1])
        fa = float(_jl_np(l, a))
        for _ in range(60):
            m = 0.5 * (a + b)
            fm = float(_jl_np(l, m))
            if fa * fm <= 0.0:
                b = m
            else:
                a, fa = m, fm
        roots.append(0.5 * (a + b))
    return np.asarray(roots)


_ZEROS = np.stack([_bessel_zeros(l, N_RADIAL) for l in range(MAX_L + 1)])
_ZFLAT = _ZEROS.reshape(1, 4 * N_RADIAL).astype(np.float32)

_LOFF = [0, 9, 36, 81]
_NRAD = 4 * N_RADIAL

_SELY = np.zeros((16, NK), np.float32)
_SELR = np.zeros((_NRAD, NK), np.float32)
_lmoff = [0, 1, 4, 9]
for _l in range(MAX_L + 1):
    for _m in range(2 * _l + 1):
        for _n in range(N_RADIAL):
            _k = _LOFF[_l] + _m * N_RADIAL + _n
            _SELY[_lmoff[_l] + _m, _k] = 1.0
            _SELR[_l * N_RADIAL + _n, _k] = 1.0


_AROWS = N_EDGES // 128
_ABR = 10
_AEB = _ABR * 128


def _edge_feat_body(x_ref, y_ref, z_ref, zcol_ref, sely_ref, selr_ref,
                    out_ref):
    xs = x_ref[0]
    ys = y_ref[0]
    zs = z_ref[0]
    r2 = xs * xs + ys * ys + zs * zs + 1e-20
    r = jnp.sqrt(r2)
    ux = xs / r
    uy = ys / r
    uz = zs / r
    t = jnp.clip((r - (CUTOFF - WIDTH)) / WIDTH, 0.0, 1.0)
    fc = 0.5 * (1.0 + jnp.cos(np.pi * t))
    xx = r / CUTOFF
    zc = zcol_ref[...]

    for rb in range(_ABR):
        x1 = ux[rb:rb + 1, :]
        y1 = uy[rb:rb + 1, :]
        z1 = uz[rb:rb + 1, :]
        one = jnp.ones_like(x1)
        yrows = [
            0.28209479177387814 * one,
            0.4886025119029199 * y1,
            0.4886025119029199 * z1,
            0.4886025119029199 * x1,
            1.0925484305920792 * x1 * y1,
            1.0925484305920792 * y1 * z1,
            0.31539156525252005 * (3.0 * z1 * z1 - 1.0),
            1.0925484305920792 * x1 * z1,
            0.5462742152960396 * (x1 * x1 - y1 * y1),
            0.5900435899266435 * y1 * (3.0 * x1 * x1 - y1 * y1),
            2.890611442640554 * x1 * y1 * z1,
            0.4570457994644658 * y1 * (5.0 * z1 * z1 - 1.0),
            0.3731763325901154 * z1 * (5.0 * z1 * z1 - 3.0),
            0.4570457994644658 * x1 * (5.0 * z1 * z1 - 1.0),
            1.445305721320277 * z1 * (x1 * x1 - y1 * y1),
            0.5900435899266435 * x1 * (x1 * x1 - 3.0 * y1 * y1),
        ]
        Yt = jnp.concatenate(yrows, axis=0)

        arg = jnp.maximum(zc * xx[rb:rb + 1, :], 1e-6)
        s = jnp.sin(arg)
        c = jnp.cos(arg)
        a0 = arg[0:9, :]
        s0 = s[0:9, :]
        a1 = arg[9:18, :]
        s1 = s[9:18, :]
        c1 = c[9:18, :]
        a2 = arg[18:27, :]
        s2 = s[18:27, :]
        c2 = c[18:27, :]
        a3 = arg[27:36, :]
        s3 = s[27:36, :]
        c3 = c[27:36, :]
        j0 = s0 / a0
        j1 = s1 / a1**2 - c1 / a1
        j2 = (3.0 / a2**3 - 1.0 / a2) * s2 - 3.0 * c2 / a2**2
        j3 = ((15.0 / a3**4 - 6.0 / a3**2) * s3
              - (15.0 / a3**3 - 1.0 / a3) * c3)
        Rt = jnp.concatenate([j0, j1, j2, j3], axis=0) * fc[rb:rb + 1, :]

        Ye = Yt.T
        Re = Rt.T
        G = (jnp.dot(Ye, sely_ref[...], preferred_element_type=jnp.float32)
             * jnp.dot(Re, selr_ref[...],
                       preferred_element_type=jnp.float32))
        out_ref[rb * 128:(rb + 1) * 128, :] = G


def _edge_features(R_ij):
    grid = _AROWS // _ABR
    planes = R_ij.T.reshape(3, grid, _ABR, 128)
    out = pl.pallas_call(
        _edge_feat_body,
        grid=(grid,),
        compiler_params=pltpu.CompilerParams(
            fuse_transposed_lhs_in_matmul=True),
        in_specs=[
            pl.BlockSpec((1, _ABR, 128), lambda ib: (ib, 0, 0)),
            pl.BlockSpec((1, _ABR, 128), lambda ib: (ib, 0, 0)),
            pl.BlockSpec((1, _ABR, 128), lambda ib: (ib, 0, 0)),
            pl.BlockSpec((_NRAD, 1), lambda ib: (0, 0)),
            pl.BlockSpec((16, NK), lambda ib: (0, 0)),
            pl.BlockSpec((_NRAD, NK), lambda ib: (0, 0)),
        ],
        out_specs=pl.BlockSpec((_AEB, NK), lambda ib: (ib, 0)),
        out_shape=jax.ShapeDtypeStruct((N_EDGES, NK), jnp.float32),
    )(planes[0], planes[1], planes[2],
      jnp.asarray(_ZFLAT.reshape(_NRAD, 1)), jnp.asarray(_SELY),
      jnp.asarray(_SELR))
    return out


_SC_NC = 2
_SC_NT = 16
_PASSES = -(-NGROUP // _SC_NC)
_CHUNK = 128
_NCHUNKS = N_EDGES // _CHUNK
_ROWS_PT = NSEG // _SC_NT
_DCH = 1000


def _sc_scatter(g2, i, j, species, zrows):
    mesh = plsc.VectorSubcoreMesh(core_axis_name="c", subcore_axis_name="s")

    @functools.partial(
        pl.kernel,
        out_type=jax.ShapeDtypeStruct((NGROUP * NSEG, GW), jnp.float32),
        name="soap_sc_scatter",
        mesh=mesh,
        compiler_params=pltpu.CompilerParams(
            needs_layout_passes=False, use_tc_tiling_on_sc=False),
        scratch_types=[
            pltpu.VMEM_SHARED((NSEG, GW), jnp.float32),
            pltpu.VMEM((N_NODES,), jnp.int32),
            pltpu.VMEM((_DCH, GW), jnp.float32),
            pltpu.VMEM((_CHUNK, GW), jnp.float32),
            pltpu.VMEM((_CHUNK, GW), jnp.float32),
            pltpu.VMEM((_CHUNK,), jnp.int32),
            pltpu.VMEM((_CHUNK,), jnp.int32),
            pltpu.VMEM((_CHUNK,), jnp.int32),
            pltpu.VMEM((_CHUNK,), jnp.int32),
            pltpu.VMEM((_CHUNK,), jnp.int32),
            pltpu.VMEM((_CHUNK,), jnp.int32),
            pltpu.SemaphoreType.DMA,
            pltpu.SemaphoreType.DMA,
        ],
    )
    def scatter_kernel(g_hbm, i_hbm, j_hbm, sp_hbm, z_hbm, h_hbm,
                       slab, spec_v, zbuf, gb0, gb1, ib0, ib1, jb0, jb1,
                       kb0, kb1, sem0, sem1):
        c = lax.axis_index("c")
        t = lax.axis_index("s")
        gbuf = (gb0, gb1)
        ibuf = (ib0, ib1)
        jbuf = (jb0, jb1)
        kbuf = (kb0, kb1)
        sems = (sem0, sem1)

        pltpu.sync_copy(sp_hbm, spec_v)
        pltpu.sync_copy(z_hbm, zbuf)

        q0 = (t * _NCHUNKS) // _SC_NT
        q1 = ((t + 1) * _NCHUNKS) // _SC_NT
        nq = q1 - q0
        r0 = t * _ROWS_PT

        def fire(gcol, q, slot):
            e0 = pl.multiple_of(q * _CHUNK, _CHUNK)
            pltpu.async_copy(
                g_hbm.at[pl.ds(e0, _CHUNK), pl.ds(gcol, GW)], gbuf[slot],
                sems[slot])
            pltpu.async_copy(i_hbm.at[pl.ds(e0, _CHUNK)], ibuf[slot],
                             sems[slot])
            pltpu.async_copy(j_hbm.at[pl.ds(e0, _CHUNK)], jbuf[slot],
                             sems[slot])

        def drain(slot):
            pltpu.make_async_copy(
                g_hbm.at[pl.ds(0, _CHUNK), pl.ds(0, GW)], gbuf[slot],
                sems[slot]).wait()
            pltpu.make_async_copy(
                i_hbm.at[pl.ds(0, _CHUNK)], ibuf[slot], sems[slot]).wait()
            pltpu.make_async_copy(
                j_hbm.at[pl.ds(0, _CHUNK)], jbuf[slot], sems[slot]).wait()

        for p in range(_PASSES):
            g = p * _SC_NC + c

            @pl.when(g < NGROUP)
            def _pass_body(g=g):
                gcol = g * GW
                hrow_base = g * NSEG

                for dch in range(_ROWS_PT // _DCH):
                    zoff = pl.multiple_of(r0 + dch * _DCH, 8)
                    pltpu.sync_copy(zbuf, slab.at[pl.ds(zoff, _DCH), :])
                plsc.subcore_barrier()

                def chunk_iter(s, slot):
                    drain(slot)

                    @pl.when(s + 1 < nq)
                    def _():
                        fire(gcol, q0 + s + 1, 1 - slot)

                    for u in range(_CHUNK // 16):
                        iv = ibuf[slot][pl.ds(u * 16, 16)]
                        jv = jbuf[slot][pl.ds(u * 16, 16)]
                        sv = plsc.load_gather(spec_v, [jv])
                        kbuf[slot][pl.ds(u * 16, 16)] = iv * N_SPECIES + sv
                    pltpu.sync_copy(gbuf[slot], slab.at[kbuf[slot]],
                                    add=True)

                fire(gcol, q0, 0)

                def loop_body(s, carry):
                    even = lax.rem(s, 2) == 0

                    @pl.when(even)
                    def _():
                        chunk_iter(s, 0)

                    @pl.when(jnp.logical_not(even))
                    def _():
                        chunk_iter(s, 1)

                    return carry

                lax.fori_loop(0, nq, loop_body, 0)
                plsc.subcore_barrier()

                for dch in range(_ROWS_PT // _DCH):
                    rr = pl.multiple_of(r0 + dch * _DCH, 8)
                    ro = pl.multiple_of(hrow_base + r0 + dch * _DCH, 8)
                    pltpu.sync_copy(slab.at[pl.ds(rr, _DCH), :],
                                    h_hbm.at[pl.ds(ro, _DCH), :])

    return scatter_kernel(g2, i, j, species, zrows)


def _segment_accumulate(g2, i, j, species):
    zrows = jnp.zeros((_DCH, GW), jnp.float32)
    return _sc_scatter(g2, i.astype(jnp.int32), j.astype(jnp.int32),
                       species.astype(jnp.int32), zrows)


_NB = 128

_TREP = np.zeros((36, 1296), np.float32)
_TTILE = np.zeros((36, 1296), np.float32)
for _a in range(36):
    for _b in range(36):
        _TREP[_a, _a * 36 + _b] = 1.0
        _TTILE[_b, _a * 36 + _b] = 1.0

_lm_of = []
for _l in range(MAX_L + 1):
    for _m in range(2 * _l + 1):
        _lm_of.append((_l, _m))

_TSEL = np.zeros((N_PSEUDO, NGROUP * N_SPECIES * GW, 16 * 36), np.float32)
for _g in range(NGROUP):
    for _s in range(N_SPECIES):
        for _k in range(GW):
            _row = _g * 128 + _s * GW + _k
            _kg = _g * GW + _k
            for _lm in range(16):
                _l, _m = _lm_of[_lm]
                _n = _kg - (_LOFF[_l] + _m * N_RADIAL)
                if 0 <= _n < N_RADIAL:
                    for _p in range(N_PSEUDO):
                        _TSEL[_p, _row, _lm * 36 + _p * N_RADIAL + _n] = 1.0


def _species_lane_weights(W_species):
    return jnp.stack([
        jnp.tile(jnp.repeat(W_species[:, p], GW), NGROUP)
        for p in range(N_PSEUDO)])


def _power_body(w_ref, wvec_ref, tsel_ref, trep_ref, ttile_ref, *refs):
    h_refs = refs[:NGROUP]
    out_ref = refs[NGROUP]
    del w_ref
    Hcat = jnp.concatenate([h[0] for h in h_refs], axis=1)
    V_all = None
    for p in range(N_PSEUDO):
        hw = Hcat * wvec_ref[p:p + 1, :]
        part = jnp.dot(hw, tsel_ref[p],
                       preferred_element_type=jnp.float32)
        V_all = part if V_all is None else V_all + part

    Tr = trep_ref[...]
    Tt = ttile_ref[...]
    vst = []
    for l in range(MAX_L + 1):
        lm0 = _lmoff[l]
        vst.append(jnp.concatenate(
            [V_all[:, (lm0 + m) * 36:(lm0 + m + 1) * 36]
             for m in range(2 * l + 1)], axis=0))
    vrs = [jnp.dot(v, Tr, preferred_element_type=jnp.float32) for v in vst]
    vts = [jnp.dot(v, Tt, preferred_element_type=jnp.float32) for v in vst]
    outcols = []
    for l in range(MAX_L + 1):
        prod = vrs[l] * vts[l]
        acc = prod.reshape(2 * l + 1, _NB, 1296).sum(axis=0)
        outcols.append(acc)
    out_ref[...] = jnp.concatenate(outcols, axis=1)


def _power_spectrum_tc(h2, W_species):
    h3 = h2.reshape(NGROUP, N_NODES, N_SPECIES * GW)
    grid = pl.cdiv(N_NODES, _NB)

    def _hmap(g):
        return lambda ib: (g, ib, 0)

    return pl.pallas_call(
        _power_body,
        grid=(grid,),
        in_specs=[pl.BlockSpec(memory_space=pltpu.SMEM),
                  pl.BlockSpec((N_PSEUDO, NGROUP * 128), lambda ib: (0, 0)),
                  pl.BlockSpec((N_PSEUDO, NGROUP * 128, 16 * 36),
                               lambda ib: (0, 0, 0)),
                  pl.BlockSpec((36, 1296), lambda ib: (0, 0)),
                  pl.BlockSpec((36, 1296), lambda ib: (0, 0))]
        + [pl.BlockSpec((1, _NB, N_SPECIES * GW), _hmap(g))
           for g in range(NGROUP)],
        out_specs=pl.BlockSpec((_NB, 36 * 36 * 4), lambda ib: (ib, 0)),
        out_shape=jax.ShapeDtypeStruct((N_NODES, 36 * 36 * 4), jnp.float32),
    )(W_species, _species_lane_weights(W_species), jnp.asarray(_TSEL),
      jnp.asarray(_TREP), jnp.asarray(_TTILE), *([h3] * NGROUP))


def kernel(R_ij, i, j, species, structures, centers, W_species):
    g2 = _edge_features(R_ij)
    h2 = _segment_accumulate(g2, i, j, species)
    return _power_spectrum_tc(h2, W_species)
